# baseline (speedup 1.0000x reference)
import functools
import jax
import jax.numpy as jnp
from jax import lax
from jax.experimental import pallas as pl
from jax.experimental.pallas import tpu as pltpu
from jax.experimental.pallas import tpu_sc as plsc

_NC = 2
_NS = 16
_NW = _NC * _NS
_CS = 32
_UNROLL = 16


def _sc_body(batch, seq_len, embed, x_hbm, pos_hbm, out_hbm,
             posbuf, xb0, xb1, ld0, ld1, st0, st1, pld):
    rows_per_w = seq_len // _NW
    n_chunks = rows_per_w // _CS
    n_steps = n_chunks * batch
    span = _CS * embed
    wid = lax.axis_index("s") * _NC + lax.axis_index("c")
    wbase = wid * rows_per_w * embed
    xbufs = (xb0, xb1)
    lds = (ld0, ld1)
    sts = (st0, st1)

    def start_xload(i):
        c, b = divmod(i, batch)
        return pltpu.async_copy(
            x_hbm.at[b, pl.ds(wbase + c * span, span)], xbufs[i % 2], lds[i % 2]
        )

    pos_desc = pltpu.async_copy(pos_hbm.at[pl.ds(wbase, span)], posbuf, pld)
    x_descs = {0: start_xload(0)}
    st_descs = {}
    for i in range(n_steps):
        c, b = divmod(i, batch)
        k = i % 2
        if i + 1 < n_steps:
            if i >= 1:
                st_descs[i - 1].wait()
            x_descs[i + 1] = start_xload(i + 1)
        if b == 0:
            pos_desc.wait()
        x_descs[i].wait()
        xb = xbufs[k]

        def group_add(g, carry, xb=xb):
            base = g * (_UNROLL * 16)
            for u in range(_UNROLL):
                sl = pl.ds(base + u * 16, 16)
                plsc.addupdate(xb.at[sl], posbuf[sl])
            return carry

        lax.fori_loop(0, span // (_UNROLL * 16), group_add, 0)
        if b == batch - 1 and c + 1 < n_chunks:
            pos_desc = pltpu.async_copy(
                pos_hbm.at[pl.ds(wbase + (c + 1) * span, span)], posbuf, pld
            )
        st_descs[i] = pltpu.async_copy(
            xb, out_hbm.at[b, pl.ds(wbase + c * span, span)], sts[k]
        )
    st_descs[n_steps - 2].wait()
    st_descs[n_steps - 1].wait()


def kernel(x, pos_table):
    batch, seq_len, embed = x.shape
    mesh = plsc.VectorSubcoreMesh(core_axis_name="c", subcore_axis_name="s")
    run = pl.kernel(
        functools.partial(_sc_body, batch, seq_len, embed),
        out_type=jax.ShapeDtypeStruct((batch, seq_len * embed), x.dtype),
        mesh=mesh,
        scratch_types=[
            pltpu.VMEM((_CS * embed,), jnp.float32),
            pltpu.VMEM((_CS * embed,), jnp.float32),
            pltpu.VMEM((_CS * embed,), jnp.float32),
            pltpu.SemaphoreType.DMA,
            pltpu.SemaphoreType.DMA,
            pltpu.SemaphoreType.DMA,
            pltpu.SemaphoreType.DMA,
            pltpu.SemaphoreType.DMA,
        ],
    )
    out = run(x.reshape(batch, seq_len * embed), pos_table.reshape(-1))
    return out.reshape(batch, seq_len, embed)


# speedup vs baseline: 1.3096x; 1.3096x over previous
import functools
import jax
import jax.numpy as jnp
from jax import lax
from jax.experimental import pallas as pl
from jax.experimental.pallas import tpu as pltpu
from jax.experimental.pallas import tpu_sc as plsc

_NC = 2
_NS = 16
_NW = _NC * _NS
_CS = 32
_UNROLL = 16


def _sc_body(batch, seq_len, embed, x_hbm, pos_hbm, out_hbm,
             posbuf, xb0, xb1, ld0, ld1, st0, st1, pld):
    rows_per_w = seq_len // _NW
    n_chunks = rows_per_w // _CS
    n_steps = n_chunks * batch
    gpr = embed // (_UNROLL * 16)
    wid = lax.axis_index("s") * _NC + lax.axis_index("c")
    wbase = wid * rows_per_w
    xbufs = (xb0, xb1)
    lds = (ld0, ld1)
    sts = (st0, st1)

    def start_xload(i):
        c, b = divmod(i, batch)
        return pltpu.async_copy(
            x_hbm.at[b, pl.ds(wbase + c * _CS, _CS)], xbufs[i % 2], lds[i % 2]
        )

    pos_desc = pltpu.async_copy(pos_hbm.at[pl.ds(wbase, _CS)], posbuf, pld)
    x_descs = {0: start_xload(0)}
    st_descs = {}
    for i in range(n_steps):
        c, b = divmod(i, batch)
        k = i % 2
        if i + 1 < n_steps:
            if i >= 1:
                st_descs[i - 1].wait()
            x_descs[i + 1] = start_xload(i + 1)
        if b == 0:
            pos_desc.wait()
        x_descs[i].wait()
        xb = xbufs[k]

        def group_add(g, carry, xb=xb):
            r = g // gpr
            col = (g % gpr) * (_UNROLL * 16)
            for u in range(_UNROLL):
                sl = pl.ds(col + u * 16, 16)
                plsc.addupdate(xb.at[r, sl], posbuf[r, sl])
            return carry

        lax.fori_loop(0, _CS * gpr, group_add, 0)
        if b == batch - 1 and c + 1 < n_chunks:
            pos_desc = pltpu.async_copy(
                pos_hbm.at[pl.ds(wbase + (c + 1) * _CS, _CS)], posbuf, pld
            )
        st_descs[i] = pltpu.async_copy(
            xb, out_hbm.at[b, pl.ds(wbase + c * _CS, _CS)], sts[k]
        )
    st_descs[n_steps - 2].wait()
    st_descs[n_steps - 1].wait()


def kernel(x, pos_table):
    batch, seq_len, embed = x.shape
    mesh = plsc.VectorSubcoreMesh(core_axis_name="c", subcore_axis_name="s")
    run = pl.kernel(
        functools.partial(_sc_body, batch, seq_len, embed),
        out_type=jax.ShapeDtypeStruct((batch, seq_len, embed), x.dtype),
        mesh=mesh,
        scratch_types=[
            pltpu.VMEM((_CS, embed), jnp.float32),
            pltpu.VMEM((_CS, embed), jnp.float32),
            pltpu.VMEM((_CS, embed), jnp.float32),
            pltpu.SemaphoreType.DMA,
            pltpu.SemaphoreType.DMA,
            pltpu.SemaphoreType.DMA,
            pltpu.SemaphoreType.DMA,
            pltpu.SemaphoreType.DMA,
        ],
    )
    return run(x, pos_table)


# speedup vs baseline: 2.4103x; 1.8405x over previous
import functools
import jax
import jax.numpy as jnp
from jax import lax
from jax.experimental import pallas as pl
from jax.experimental.pallas import tpu as pltpu
from jax.experimental.pallas import tpu_sc as plsc

_NC = 2
_NS = 16
_NW = _NC * _NS
_CS = 32
_UNROLL = 16


def _sc_body(batch, seq_len, embed, x_hbm, pos_hbm, out_hbm,
             posbuf, xb0, xb1, ld0, ld1, st0, st1, pld):
    rows_per_w = seq_len // _NW
    n_chunks = rows_per_w // _CS
    n_steps = n_chunks * batch
    gpr = embed // (_UNROLL * 16)
    wid = lax.axis_index("s") * _NC + lax.axis_index("c")
    wbase = wid * rows_per_w
    xbufs = (xb0, xb1)
    lds = (ld0, ld1)
    sts = (st0, st1)

    def start_xload(i):
        c, b = divmod(i, batch)
        return pltpu.async_copy(
            x_hbm.at[b, pl.ds(wbase + c * _CS, _CS)], xbufs[i % 2], lds[i % 2]
        )

    pos_desc = pltpu.async_copy(pos_hbm.at[pl.ds(wbase, _CS)], posbuf, pld)
    x_descs = {0: start_xload(0)}
    st_descs = {}
    for i in range(n_steps):
        c, b = divmod(i, batch)
        k = i % 2
        if i + 1 < n_steps:
            if i >= 1:
                st_descs[i - 1].wait()
            x_descs[i + 1] = start_xload(i + 1)
        if b == 0:
            pos_desc.wait()
        x_descs[i].wait()
        xb = xbufs[k]

        def group_add(g, carry, xb=xb):
            r = g // gpr
            colbase = (g % gpr) * (_UNROLL * 16)
            for p in range(_UNROLL // 8):
                cols = [colbase + (p * 8 + u) * 16 for u in range(8)]
                pv = [posbuf[r, pl.ds(c0, 16)] for c0 in cols]
                for c0, v in zip(cols, pv):
                    plsc.addupdate(xb.at[r, pl.ds(c0, 16)], v)
            return carry

        lax.fori_loop(0, _CS * gpr, group_add, 0)
        if b == batch - 1 and c + 1 < n_chunks:
            pos_desc = pltpu.async_copy(
                pos_hbm.at[pl.ds(wbase + (c + 1) * _CS, _CS)], posbuf, pld
            )
        st_descs[i] = pltpu.async_copy(
            xb, out_hbm.at[b, pl.ds(wbase + c * _CS, _CS)], sts[k]
        )
    st_descs[n_steps - 2].wait()
    st_descs[n_steps - 1].wait()


def kernel(x, pos_table):
    batch, seq_len, embed = x.shape
    mesh = plsc.VectorSubcoreMesh(core_axis_name="c", subcore_axis_name="s")
    run = pl.kernel(
        functools.partial(_sc_body, batch, seq_len, embed),
        out_type=jax.ShapeDtypeStruct((batch, seq_len, embed), x.dtype),
        mesh=mesh,
        scratch_types=[
            pltpu.VMEM((_CS, embed), jnp.float32),
            pltpu.VMEM((_CS, embed), jnp.float32),
            pltpu.VMEM((_CS, embed), jnp.float32),
            pltpu.SemaphoreType.DMA,
            pltpu.SemaphoreType.DMA,
            pltpu.SemaphoreType.DMA,
            pltpu.SemaphoreType.DMA,
            pltpu.SemaphoreType.DMA,
        ],
    )
    return run(x, pos_table)
